# baseline (speedup 1.0000x reference)
import functools

import jax
import jax.numpy as jnp
from jax import lax
from jax.experimental import pallas as pl
from jax.experimental.pallas import tpu as pltpu
from jax.experimental.pallas import tpu_sc as plsc

D = 1000
NC = 2
NS = 16
NW = NC * NS


@jax.jit
def _gather_rows(idx, table):
    b, t = idx.shape
    b_per_w = b // NW
    n_pairs = b_per_w // 2
    mesh = plsc.VectorSubcoreMesh(
        core_axis_name="c", subcore_axis_name="s", num_cores=NC, num_subcores=NS
    )

    @functools.partial(
        pl.kernel,
        mesh=mesh,
        out_type=jax.ShapeDtypeStruct((b, t, D), jnp.float32),
        scratch_types=[
            pltpu.VMEM((b_per_w, t), jnp.int32),
            pltpu.VMEM((2, t, D), jnp.float32),
            [pltpu.SemaphoreType.DMA] * 2,
        ],
        compiler_params=pltpu.CompilerParams(use_tc_tiling_on_sc=False),
    )
    def k(idx_hbm, table_hbm, out_hbm, idx_v, rows_v, gsems):
        wid = lax.axis_index("s") * NC + lax.axis_index("c")
        base = wid * b_per_w
        pltpu.sync_copy(idx_hbm.at[pl.ds(base, b_per_w)], idx_v)

        def gather_cp(c, s):
            return pltpu.make_async_copy(
                table_hbm.at[idx_v.at[c]], rows_v.at[s], gsems[s]
            )

        def write_out(c, s):
            pltpu.sync_copy(rows_v.at[s], out_hbm.at[base + c])

        gather_cp(0, 0).start()
        gather_cp(1, 1).start()

        def body(p, carry):
            c = 2 * p
            gather_cp(c, 0).wait()
            write_out(c, 0)
            gather_cp(c + 2, 0).start()
            gather_cp(c + 1, 1).wait()
            write_out(c + 1, 1)
            gather_cp(c + 3, 1).start()
            return carry

        lax.fori_loop(0, n_pairs - 1, body, 0)

        c = b_per_w - 2
        gather_cp(c, 0).wait()
        write_out(c, 0)
        gather_cp(c + 1, 1).wait()
        write_out(c + 1, 1)

    return k(idx, table)


def kernel(idx, table):
    return _gather_rows(idx, table)


# speedup vs baseline: 1.1183x; 1.1183x over previous
import functools

import jax
import jax.numpy as jnp
from jax import lax
from jax.experimental import pallas as pl
from jax.experimental.pallas import tpu as pltpu
from jax.experimental.pallas import tpu_sc as plsc

D = 1000
NC = 2
NS = 16
NW = NC * NS
NU = 400
PC = 32
MAXU = 13


@jax.jit
def _gather_tiled(idxT_flat, table):
    mesh = plsc.VectorSubcoreMesh(
        core_axis_name="c", subcore_axis_name="s", num_cores=NC, num_subcores=NS
    )

    @functools.partial(
        pl.kernel,
        mesh=mesh,
        out_type=jax.ShapeDtypeStruct((50, 125, 8, 8, 128), jnp.float32),
        scratch_types=[
            pltpu.VMEM((MAXU * 128,), jnp.int32),
            pltpu.VMEM((2, PC, D), jnp.float32),
            pltpu.VMEM((2, 125, 8, PC), jnp.float32),
            [pltpu.SemaphoreType.DMA] * 2,
            [pltpu.SemaphoreType.DMA] * 2,
        ],
        compiler_params=pltpu.CompilerParams(
            use_tc_tiling_on_sc=False, needs_layout_passes=False
        ),
    )
    def k(idx_hbm, table_hbm, out_hbm, idx_v, gbuf, tbuf, gsems, wsems):
        w = lax.axis_index("s") * NC + lax.axis_index("c")
        nu = 12 + (w < 16).astype(jnp.int32)
        u0 = 12 * w + jnp.minimum(w, 16)
        pltpu.sync_copy(idx_hbm.at[pl.ds(128 * u0, MAXU * 128)], idx_v)

        rows = [jnp.arange(16, dtype=jnp.int32), jnp.arange(16, dtype=jnp.int32) + 16]

        def gather_cp(P, s):
            return pltpu.make_async_copy(
                table_hbm.at[idx_v.at[pl.ds(PC * P, PC)]], gbuf.at[s], gsems[s]
            )

        def write_cp(P, s):
            u = u0 + lax.shift_right_logical(P, 2)
            p = lax.bitwise_and(P, 3)
            t = lax.shift_right_logical(u, 3)
            j = lax.bitwise_and(u, 7)
            return pltpu.make_async_copy(
                tbuf.at[s],
                out_hbm.at[t, :, j, :, pl.ds(PC * p, PC)],
                wsems[s],
            )

        def transpose(s):
            src = gbuf.at[s]
            dst = tbuf.at[s]

            def tr_body(r, carry):
                for s8 in range(8):
                    col = jnp.full((16,), 8 * r + s8, dtype=jnp.int32)
                    for h in range(2):
                        vals = plsc.load_gather(src, [rows[h], col])
                        dst[r, s8, pl.ds(16 * h, 16)] = vals
                return carry

            lax.fori_loop(0, 125, tr_body, 0)

        def piece(P, s, first, last):
            gather_cp(P, s).wait()
            if not first:
                write_cp(P - 2, s).wait()
            transpose(s)
            write_cp(P, s).start()
            if not last:
                gather_cp(P + 2, s).start()

        gather_cp(0, 0).start()
        gather_cp(1, 1).start()
        piece(0, 0, True, False)
        piece(1, 1, True, False)

        def body(p2, carry):
            piece(2 * p2, 0, False, False)
            piece(2 * p2 + 1, 1, False, False)
            return carry

        lax.fori_loop(1, 2 * nu - 1, body, 0)

        last = 4 * nu - 2
        piece(last, 0, False, True)
        piece(last + 1, 1, False, True)
        write_cp(last, 0).wait()
        write_cp(last + 1, 1).wait()

    return k(idxT_flat, table)


def kernel(idx, table):
    b, t = idx.shape
    idxT_flat = jnp.pad(idx.T.reshape(-1), (0, 128))
    p5 = _gather_tiled(idxT_flat, table)
    b5 = jnp.transpose(p5, (2, 4, 0, 1, 3))
    return b5.reshape(b, t, D)


# speedup vs baseline: 1.3136x; 1.1746x over previous
import functools

import jax
import jax.numpy as jnp
from jax import lax
from jax.experimental import pallas as pl
from jax.experimental.pallas import tpu as pltpu
from jax.experimental.pallas import tpu_sc as plsc

D = 1000
NC = 2
NS = 16
NW = NC * NS
NU = 400
PC = 16
MAXU = 13
NG = 62


@jax.jit
def _gather_tiled(idxT_flat, table):
    mesh = plsc.VectorSubcoreMesh(
        core_axis_name="c", subcore_axis_name="s", num_cores=NC, num_subcores=NS
    )

    @functools.partial(
        pl.kernel,
        mesh=mesh,
        out_type=jax.ShapeDtypeStruct((50, 125, 8, 8, 128), jnp.float32),
        scratch_types=[
            pltpu.VMEM((MAXU * 128,), jnp.int32),
            pltpu.VMEM((2, PC, D), jnp.float32),
            pltpu.VMEM((2, 125, 8, PC + 1), jnp.float32),
            [pltpu.SemaphoreType.DMA] * 2,
            [pltpu.SemaphoreType.DMA] * 2,
        ],
        compiler_params=pltpu.CompilerParams(
            use_tc_tiling_on_sc=False, needs_layout_passes=False
        ),
    )
    def k(idx_hbm, table_hbm, out_hbm, idx_v, gbuf, tbuf, gsems, wsems):
        w = lax.axis_index("s") * NC + lax.axis_index("c")
        nu = 12 + (w < 16).astype(jnp.int32)
        u0 = 12 * w + jnp.minimum(w, 16)
        pltpu.sync_copy(idx_hbm.at[pl.ds(128 * u0, MAXU * 128)], idx_v)

        iota = jnp.arange(16, dtype=jnp.int32)
        cvecs = [jnp.full((16,), c, dtype=jnp.int32) for c in range(PC)]

        def gather_cp(P, s):
            return pltpu.make_async_copy(
                table_hbm.at[idx_v.at[pl.ds(PC * P, PC)]], gbuf.at[s], gsems[s]
            )

        def write_cp(P, s):
            u = u0 + lax.shift_right_logical(P, 3)
            p = lax.bitwise_and(P, 7)
            t = lax.shift_right_logical(u, 3)
            j = lax.bitwise_and(u, 7)
            return pltpu.make_async_copy(
                tbuf.at[s, :, :, pl.ds(0, PC)],
                out_hbm.at[t, :, j, :, pl.ds(PC * p, PC)],
                wsems[s],
            )

        def tr_group(src, dst, d0):
            dvec = iota + d0
            rvec = lax.shift_right_logical(dvec, 3)
            svec = lax.bitwise_and(dvec, 7)
            for c in range(PC):
                vals = src[c, pl.ds(d0, 16)]
                plsc.store_scatter(dst, [rvec, svec, cvecs[c]], vals)

        def transpose(s):
            src = gbuf.at[s]
            dst = tbuf.at[s]

            def tr_body(g, carry):
                tr_group(src, dst, 16 * g)
                return carry

            lax.fori_loop(0, NG, tr_body, 0)
            tr_group(src, dst, D - 16)

        def piece(P, s, first, last):
            gather_cp(P, s).wait()
            if not first:
                write_cp(P - 2, s).wait()
            transpose(s)
            write_cp(P, s).start()
            if not last:
                gather_cp(P + 2, s).start()

        gather_cp(0, 0).start()
        gather_cp(1, 1).start()
        piece(0, 0, True, False)
        piece(1, 1, True, False)

        def body(p2, carry):
            piece(2 * p2, 0, False, False)
            piece(2 * p2 + 1, 1, False, False)
            return carry

        lax.fori_loop(1, 4 * nu - 1, body, 0)

        last = 8 * nu - 2
        piece(last, 0, False, True)
        piece(last + 1, 1, False, True)
        write_cp(last, 0).wait()
        write_cp(last + 1, 1).wait()

    return k(idxT_flat, table)


def kernel(idx, table):
    b, t = idx.shape
    idxT_flat = jnp.pad(idx.T.reshape(-1), (0, 128))
    p5 = _gather_tiled(idxT_flat, table)
    b5 = jnp.transpose(p5, (2, 4, 0, 1, 3))
    return b5.reshape(b, t, D)


# speedup vs baseline: 2.7806x; 2.1168x over previous
import functools

import jax
import jax.numpy as jnp
from jax import lax
from jax.experimental import pallas as pl
from jax.experimental.pallas import tpu as pltpu
from jax.experimental.pallas import tpu_sc as plsc

D = 1000
NC = 2
NS = 16
NW = NC * NS
NU = 400
PC = 16
MAXU = 13
NG = 62


@jax.jit
def _gather_tiled(idxT_flat, table):
    mesh = plsc.VectorSubcoreMesh(
        core_axis_name="c", subcore_axis_name="s", num_cores=NC, num_subcores=NS
    )

    @functools.partial(
        pl.kernel,
        mesh=mesh,
        out_type=jax.ShapeDtypeStruct((50, 125, 8, 8, 128), jnp.float32),
        scratch_types=[
            pltpu.VMEM((MAXU * 128,), jnp.int32),
            pltpu.VMEM((2, PC, D), jnp.float32),
            pltpu.VMEM((2, 125, 8, PC + 1), jnp.float32),
            [pltpu.SemaphoreType.DMA] * 2,
            [pltpu.SemaphoreType.DMA] * 2,
        ],
        compiler_params=pltpu.CompilerParams(
            use_tc_tiling_on_sc=False, needs_layout_passes=False
        ),
    )
    def k(idx_hbm, table_hbm, out_hbm, idx_v, gbuf, tbuf, gsems, wsems):
        w = lax.axis_index("s") * NC + lax.axis_index("c")
        nu = 12 + (w < 16).astype(jnp.int32)
        u0 = 12 * w + jnp.minimum(w, 16)
        pltpu.sync_copy(idx_hbm.at[pl.ds(128 * u0, MAXU * 128)], idx_v)

        iota = jnp.arange(16, dtype=jnp.int32)
        cvecs = [jnp.full((16,), c, dtype=jnp.int32) for c in range(PC)]

        def gather_cp(P, s):
            return pltpu.make_async_copy(
                table_hbm.at[idx_v.at[pl.ds(PC * P, PC)]], gbuf.at[s], gsems[s]
            )

        def write_cp(P, s):
            u = u0 + lax.shift_right_logical(P, 3)
            p = lax.bitwise_and(P, 7)
            t = lax.shift_right_logical(u, 3)
            j = lax.bitwise_and(u, 7)
            return pltpu.make_async_copy(
                tbuf.at[s, :, :, pl.ds(0, PC)],
                out_hbm.at[t, :, j, :, pl.ds(PC * p, PC)],
                wsems[s],
            )

        def tr_group(src, dst, d0):
            dvec = iota + d0
            rvec = lax.shift_right_logical(dvec, 3)
            svec = lax.bitwise_and(dvec, 7)
            for c in range(PC):
                vals = src[c, pl.ds(d0, 16)]
                plsc.store_scatter(dst, [rvec, svec, cvecs[c]], vals)

        def transpose(s):
            src = gbuf.at[s]
            dst = tbuf.at[s]

            @plsc.parallel_loop(0, NG, unroll=2)
            def tr_body(g):
                tr_group(src, dst, 16 * g)

            tr_group(src, dst, D - 16)

        def piece(P, s, first, last):
            gather_cp(P, s).wait()
            if not first:
                write_cp(P - 2, s).wait()
            transpose(s)
            write_cp(P, s).start()
            if not last:
                gather_cp(P + 2, s).start()

        gather_cp(0, 0).start()
        gather_cp(1, 1).start()
        piece(0, 0, True, False)
        piece(1, 1, True, False)

        def body(p2, carry):
            piece(2 * p2, 0, False, False)
            piece(2 * p2 + 1, 1, False, False)
            return carry

        lax.fori_loop(1, 4 * nu - 1, body, 0)

        last = 8 * nu - 2
        piece(last, 0, False, True)
        piece(last + 1, 1, False, True)
        write_cp(last, 0).wait()
        write_cp(last + 1, 1).wait()

    return k(idxT_flat, table)


def kernel(idx, table):
    b, t = idx.shape
    idxT_flat = jnp.pad(idx.T.reshape(-1), (0, 128))
    p5 = _gather_tiled(idxT_flat, table)
    b5 = jnp.transpose(p5, (2, 4, 0, 1, 3))
    return b5.reshape(b, t, D)
